# speedup vs baseline: 3.3345x; 3.3345x over previous
import functools

import jax
import jax.numpy as jnp
from jax import lax
from jax.experimental import pallas as pl
from jax.experimental.pallas import tpu as pltpu
from jax.experimental.pallas import tpu_sc as plsc

EMB_DIM = 128
GRP = 128
CH = 4


@functools.partial(jax.jit, static_argnums=(2, 3))
def _sc_gather(weight, idx2d, n_groups, n_workers):
    groups_per_w = n_groups // n_workers
    n_iter = groups_per_w // CH
    mesh = plsc.VectorSubcoreMesh(core_axis_name="c", subcore_axis_name="s")
    nc = mesh.num_cores

    @functools.partial(
        pl.kernel,
        out_type=jax.ShapeDtypeStruct((n_groups, GRP, EMB_DIM), jnp.float32),
        mesh=mesh,
        scratch_types=[
            pltpu.VMEM((CH, GRP), jnp.int32),
            pltpu.VMEM((CH, GRP, EMB_DIM), jnp.float32),
            pltpu.SemaphoreType.DMA,
        ],
    )
    def k(table_hbm, idx_hbm, out_hbm, idx_v, rows_v, sem):
        wid = lax.axis_index("s") * nc + lax.axis_index("c")
        w_base = wid * groups_per_w

        def body(i, carry):
            g = w_base + i * CH
            pltpu.sync_copy(idx_hbm.at[pl.ds(g, CH)], idx_v)
            copies = [
                pltpu.async_copy(table_hbm.at[idx_v.at[j]], rows_v.at[j], sem)
                for j in range(CH)
            ]
            for c in copies:
                c.wait()
            pltpu.sync_copy(rows_v, out_hbm.at[pl.ds(g, CH)])
            return carry

        lax.fori_loop(0, n_iter, body, 0)

    return k(weight, idx2d)


def kernel(input, weight):
    b, s = input.shape
    n = b * s
    idx2d = input.reshape(n // GRP, GRP).astype(jnp.int32)
    out = _sc_gather(weight, idx2d, n // GRP, 32)
    return out.reshape(b, s, EMB_DIM)


# speedup vs baseline: 3.4584x; 1.0372x over previous
import functools

import jax
import jax.numpy as jnp
from jax import lax
from jax.experimental import pallas as pl
from jax.experimental.pallas import tpu as pltpu
from jax.experimental.pallas import tpu_sc as plsc

EMB_DIM = 128
GRP = 128
CH = 2
NBUF = 2


@functools.partial(jax.jit, static_argnums=(2, 3))
def _sc_gather(weight, idx2d, n_groups, n_workers):
    groups_per_w = n_groups // n_workers
    n_iter = groups_per_w // CH
    mesh = plsc.VectorSubcoreMesh(core_axis_name="c", subcore_axis_name="s")
    nc = mesh.num_cores

    @functools.partial(
        pl.kernel,
        out_type=jax.ShapeDtypeStruct((n_groups, GRP, EMB_DIM), jnp.float32),
        mesh=mesh,
        scratch_types=[
            pltpu.VMEM((groups_per_w, GRP), jnp.int32),
            pltpu.VMEM((NBUF, CH, GRP, EMB_DIM), jnp.float32),
            pltpu.SemaphoreType.DMA,
            pltpu.SemaphoreType.DMA,
            pltpu.SemaphoreType.DMA,
            pltpu.SemaphoreType.DMA,
        ],
    )
    def k(table_hbm, idx_hbm, out_hbm, idx_v, rows_v, g0, g1, o0, o1):
        gsem = (g0, g1)
        osem = (o0, o1)
        wid = lax.axis_index("s") * nc + lax.axis_index("c")
        w_base = wid * groups_per_w

        def fire_gathers(chunk_local, buf):
            for j in range(CH):
                pltpu.async_copy(
                    table_hbm.at[idx_v.at[chunk_local * CH + j]],
                    rows_v.at[buf].at[j],
                    gsem[buf],
                )

        def drain(buf, sem):
            pltpu.make_async_copy(
                out_hbm.at[pl.ds(0, CH)], rows_v.at[buf], sem
            ).wait()

        pltpu.sync_copy(idx_hbm.at[pl.ds(w_base, groups_per_w)], idx_v)
        fire_gathers(0, 0)

        def body(i, carry):
            p = lax.rem(i, 2)
            for buf in range(NBUF):

                @pl.when(p == buf)
                def _():
                    q = 1 - buf
                    drain(buf, gsem[buf])
                    pltpu.async_copy(
                        rows_v.at[buf],
                        out_hbm.at[pl.ds(w_base + i * CH, CH)],
                        osem[buf],
                    )

                    @pl.when(i > 0)
                    def _():
                        drain(q, osem[q])

                    @pl.when(i + 1 < n_iter)
                    def _():
                        fire_gathers(i + 1, q)

            return carry

        lax.fori_loop(0, n_iter, body, 0)
        drain((n_iter - 1) % 2, osem[(n_iter - 1) % 2])

    return k(weight, idx2d)


def kernel(input, weight):
    b, s = input.shape
    n = b * s
    idx2d = input.reshape(n // GRP, GRP).astype(jnp.int32)
    out = _sc_gather(weight, idx2d, n // GRP, 32)
    return out.reshape(b, s, EMB_DIM)


# speedup vs baseline: 5.5187x; 1.5957x over previous
import functools

import jax
import jax.numpy as jnp
from jax import lax
from jax.experimental import pallas as pl
from jax.experimental.pallas import tpu as pltpu
from jax.experimental.pallas import tpu_sc as plsc

EMB_DIM = 128
GRP = 128
CH = 2
NBUF = 2
SEQ_PAD = 8


@functools.partial(jax.jit, static_argnums=(3, 4, 5))
def _sc_gather(weight, idx2d, dst2d, n_groups, out_rows, n_workers):
    groups_per_w = n_groups // n_workers
    n_iter = groups_per_w // CH
    mesh = plsc.VectorSubcoreMesh(core_axis_name="c", subcore_axis_name="s")
    nc = mesh.num_cores

    @functools.partial(
        pl.kernel,
        out_type=jax.ShapeDtypeStruct((out_rows, EMB_DIM), jnp.float32),
        mesh=mesh,
        scratch_types=[
            pltpu.VMEM((groups_per_w, GRP), jnp.int32),
            pltpu.VMEM((groups_per_w, GRP), jnp.int32),
            pltpu.VMEM((NBUF, CH, GRP, EMB_DIM), jnp.float32),
            pltpu.SemaphoreType.DMA,
            pltpu.SemaphoreType.DMA,
            pltpu.SemaphoreType.DMA,
            pltpu.SemaphoreType.DMA,
        ],
    )
    def k(table_hbm, idx_hbm, dst_hbm, out_hbm, idx_v, dst_v, rows_v,
          g0, g1, o0, o1):
        gsem = (g0, g1)
        osem = (o0, o1)
        wid = lax.axis_index("s") * nc + lax.axis_index("c")
        w_base = wid * groups_per_w

        def fire_gathers(chunk_local, buf):
            for j in range(CH):
                pltpu.async_copy(
                    table_hbm.at[idx_v.at[chunk_local * CH + j]],
                    rows_v.at[buf].at[j],
                    gsem[buf],
                )

        def fire_scatters(chunk_local, buf):
            for j in range(CH):
                pltpu.async_copy(
                    rows_v.at[buf].at[j],
                    out_hbm.at[dst_v.at[chunk_local * CH + j]],
                    osem[buf],
                )

        def drain(buf, sem):
            pltpu.make_async_copy(
                table_hbm.at[idx_v.at[0]], rows_v.at[buf], sem
            ).wait()

        pltpu.sync_copy(idx_hbm.at[pl.ds(w_base, groups_per_w)], idx_v)
        pltpu.sync_copy(dst_hbm.at[pl.ds(w_base, groups_per_w)], dst_v)
        fire_gathers(0, 0)

        def body(i, carry):
            p = lax.rem(i, 2)
            for buf in range(NBUF):

                @pl.when(p == buf)
                def _():
                    q = 1 - buf
                    drain(buf, gsem[buf])
                    fire_scatters(i, buf)

                    @pl.when(i > 0)
                    def _():
                        drain(q, osem[q])

                    @pl.when(i + 1 < n_iter)
                    def _():
                        fire_gathers(i + 1, q)

            return carry

        lax.fori_loop(0, n_iter, body, 0)
        drain((n_iter - 1) % 2, osem[(n_iter - 1) % 2])

    return k(weight, idx2d, dst2d)


def kernel(input, weight):
    b, s = input.shape
    n = b * s
    s_pad = (s + SEQ_PAD - 1) // SEQ_PAD * SEQ_PAD
    idx2d = input.reshape(n // GRP, GRP).astype(jnp.int32)
    j = jnp.arange(n, dtype=jnp.int32).reshape(n // GRP, GRP)
    dst2d = (j // s) * s_pad + j % s
    out = _sc_gather(weight, idx2d, dst2d, n // GRP, b * s_pad, 32)
    return out.reshape(b, s_pad, EMB_DIM)[:, :s, :]


# speedup vs baseline: 11.9534x; 2.1660x over previous
import functools

import jax
import jax.numpy as jnp
from jax import lax
from jax.experimental import pallas as pl
from jax.experimental.pallas import tpu as pltpu
from jax.experimental.pallas import tpu_sc as plsc

EMB_DIM = 128
GRP = 128
CH = 2
NBUF = 2


@functools.partial(jax.jit, static_argnums=(2, 3))
def _sc_gather(weight, idx2d, n_groups, n_workers):
    groups_per_w = n_groups // n_workers
    n_iter = groups_per_w // CH
    mesh = plsc.VectorSubcoreMesh(core_axis_name="c", subcore_axis_name="s")
    nc = mesh.num_cores

    @functools.partial(
        pl.kernel,
        out_type=jax.ShapeDtypeStruct((n_groups, GRP, EMB_DIM), jnp.float32),
        mesh=mesh,
        scratch_types=[
            pltpu.VMEM((groups_per_w, GRP), jnp.int32),
            pltpu.VMEM((NBUF, CH, GRP, EMB_DIM), jnp.float32),
            pltpu.SemaphoreType.DMA,
            pltpu.SemaphoreType.DMA,
            pltpu.SemaphoreType.DMA,
            pltpu.SemaphoreType.DMA,
        ],
    )
    def k(table_hbm, idx_hbm, out_hbm, idx_v, rows_v, g0, g1, o0, o1):
        gsem = (g0, g1)
        osem = (o0, o1)
        wid = lax.axis_index("s") * nc + lax.axis_index("c")
        w_base = wid * groups_per_w

        def fire_gathers(chunk_local, buf):
            for j in range(CH):
                pltpu.async_copy(
                    table_hbm.at[idx_v.at[chunk_local * CH + j]],
                    rows_v.at[buf].at[j],
                    gsem[buf],
                )

        def drain(buf, sem):
            pltpu.make_async_copy(
                table_hbm.at[idx_v.at[0]], rows_v.at[buf], sem
            ).wait()

        pltpu.sync_copy(idx_hbm.at[pl.ds(w_base, groups_per_w)], idx_v)
        fire_gathers(0, 0)

        def body(i, carry):
            p = lax.rem(i, 2)
            for buf in range(NBUF):

                @pl.when(p == buf)
                def _():
                    q = 1 - buf
                    drain(buf, gsem[buf])
                    pltpu.async_copy(
                        rows_v.at[buf],
                        out_hbm.at[pl.ds(w_base + i * CH, CH)],
                        osem[buf],
                    )

                    @pl.when(i > 0)
                    def _():
                        drain(q, osem[q])

                    @pl.when(i + 1 < n_iter)
                    def _():
                        fire_gathers(i + 1, q)

            return carry

        lax.fori_loop(0, n_iter, body, 0)
        drain((n_iter - 1) % 2, osem[(n_iter - 1) % 2])

    return k(weight, idx2d)


def kernel(input, weight):
    b, s = input.shape
    n = b * s
    idx2d = input.T.reshape(n // GRP, GRP).astype(jnp.int32)
    out = _sc_gather(weight, idx2d, n // GRP, 32)
    return out.reshape(s, b, EMB_DIM).transpose(1, 0, 2)


# speedup vs baseline: 12.0065x; 1.0044x over previous
import functools

import jax
import jax.numpy as jnp
from jax import lax
from jax.experimental import pallas as pl
from jax.experimental.pallas import tpu as pltpu
from jax.experimental.pallas import tpu_sc as plsc

EMB_DIM = 128
GRP = 128
NBUF = 6
ODEPTH = 3


@functools.partial(jax.jit, static_argnums=(2, 3))
def _sc_gather(weight, idx2d, n_groups, n_workers):
    groups_per_w = n_groups // n_workers
    n_iter = groups_per_w
    gahead = NBUF - ODEPTH
    mesh = plsc.VectorSubcoreMesh(core_axis_name="c", subcore_axis_name="s")
    nc = mesh.num_cores

    @functools.partial(
        pl.kernel,
        out_type=jax.ShapeDtypeStruct((n_groups, GRP, EMB_DIM), jnp.float32),
        mesh=mesh,
        scratch_types=[
            pltpu.VMEM((groups_per_w, GRP), jnp.int32),
            pltpu.VMEM((NBUF, GRP, EMB_DIM), jnp.float32),
        ]
        + [pltpu.SemaphoreType.DMA] * (2 * NBUF),
    )
    def k(table_hbm, idx_hbm, out_hbm, idx_v, rows_v, *sems):
        gsem = sems[:NBUF]
        osem = sems[NBUF:]
        wid = lax.axis_index("s") * nc + lax.axis_index("c")
        w_base = wid * groups_per_w

        def fire_gather(chunk, slot):
            pltpu.async_copy(
                table_hbm.at[idx_v.at[chunk]], rows_v.at[slot], gsem[slot]
            )

        def drain(slot, sem):
            pltpu.make_async_copy(
                table_hbm.at[idx_v.at[0]], rows_v.at[slot], sem
            ).wait()

        pltpu.sync_copy(idx_hbm.at[pl.ds(w_base, groups_per_w)], idx_v)
        for c in range(gahead):
            fire_gather(c, c)

        def body(i, carry):
            p = lax.rem(i, NBUF)
            for slot in range(NBUF):

                @pl.when(p == slot)
                def _():
                    drain(slot, gsem[slot])
                    pltpu.async_copy(
                        rows_v.at[slot],
                        out_hbm.at[w_base + i],
                        osem[slot],
                    )
                    prev = (slot - ODEPTH) % NBUF

                    @pl.when(i >= ODEPTH)
                    def _():
                        drain(prev, osem[prev])

                    @pl.when(i + gahead < n_iter)
                    def _():
                        fire_gather(i + gahead, prev)

            return carry

        lax.fori_loop(0, n_iter, body, 0)
        for j in range(ODEPTH):
            slot = (n_iter - ODEPTH + j) % NBUF
            drain(slot, osem[slot])

    return k(weight, idx2d)


def kernel(input, weight):
    b, s = input.shape
    n = b * s
    idx2d = input.T.reshape(n // GRP, GRP).astype(jnp.int32)
    out = _sc_gather(weight, idx2d, n // GRP, 32)
    return out.reshape(s, b, EMB_DIM).transpose(1, 0, 2)
